# Initial kernel scaffold; baseline (speedup 1.0000x reference)
import jax
import jax.numpy as jnp
from jax.experimental import pallas as pl


def kernel(user_table, item_table, edge_u, edge_i, user_id, item_id, neg_item_id):
    raise NotImplementedError("write your pallas kernel here")


# speedup vs baseline: 1.0055x; 1.0055x over previous
import functools

import jax
import jax.numpy as jnp
from jax import lax
from jax.experimental import pallas as pl
from jax.experimental.pallas import tpu as pltpu

U = 25000
I = 25000
N = U + I
D = 64
E = 400000
B = 4096
LAYERS = 3
EPS = 0.1
TAU = 0.2
LMBD_SSL = 0.1
LMBD_REG = 1e-4

BR = 512
NB = B // BR


def _l2norm(x):
    return x / jnp.maximum(jnp.linalg.norm(x, axis=-1, keepdims=True), 1e-12)


def _loss_body(z1_ref, z2full_ref, z2blk_ref, ue_ref, pe_ref, ne_ref,
               ue0_ref, pe0_ref, ne0_ref,
               sslu_ref, ssli_ref, bpr_ref, reg_ref):
    p = pl.program_id(0)
    b = pl.program_id(1)

    @pl.when(jnp.logical_and(p == 0, b == 0))
    def _init():
        ue = ue_ref[...]
        pe = pe_ref[...]
        ne = ne_ref[...]
        s = jnp.sum(ue * (pe - ne), axis=-1)
        ls = jnp.minimum(s, 0.0) - jnp.log1p(jnp.exp(-jnp.abs(s)))
        bpr_ref[0, 0] = jnp.sum(ls)
        reg_ref[0, 0] = (jnp.sum(ue0_ref[...] ** 2) + jnp.sum(pe0_ref[...] ** 2)
                         + jnp.sum(ne0_ref[...] ** 2))
        sslu_ref[0, 0] = 0.0
        ssli_ref[0, 0] = 0.0

    z1 = z1_ref[0]
    z2f = z2full_ref[0]
    z2b = z2blk_ref[0]
    n1 = z1 / jnp.maximum(jnp.sqrt(jnp.sum(z1 * z1, -1, keepdims=True)), 1e-12)
    n2f = z2f / jnp.maximum(jnp.sqrt(jnp.sum(z2f * z2f, -1, keepdims=True)), 1e-12)
    n2b = z2b / jnp.maximum(jnp.sqrt(jnp.sum(z2b * z2b, -1, keepdims=True)), 1e-12)
    pos = jnp.sum(n1 * n2b, axis=-1) / TAU
    logits = lax.dot_general(n1, n2f, (((1,), (1,)), ((), ())),
                             preferred_element_type=jnp.float32) / TAU
    m = jnp.max(logits, axis=1)
    lse = m + jnp.log(jnp.sum(jnp.exp(logits - m[:, None]), axis=1))
    val = jnp.sum(lse - pos)

    @pl.when(p == 0)
    def _accu():
        sslu_ref[0, 0] += val

    @pl.when(p == 1)
    def _acci():
        ssli_ref[0, 0] += val


@jax.jit
def _loss_parts(z1s, z2s, ue, pe, ne, ue0, pe0, ne0):
    scalar = jax.ShapeDtypeStruct((1, 1), jnp.float32)
    smem = pl.BlockSpec(memory_space=pltpu.SMEM)
    grid = (2, NB)
    return pl.pallas_call(
        _loss_body,
        grid=grid,
        in_specs=[
            pl.BlockSpec((1, BR, D), lambda p, b: (p, b, 0)),
            pl.BlockSpec((1, B, D), lambda p, b: (p, 0, 0)),
            pl.BlockSpec((1, BR, D), lambda p, b: (p, b, 0)),
            pl.BlockSpec((B, D), lambda p, b: (0, 0)),
            pl.BlockSpec((B, D), lambda p, b: (0, 0)),
            pl.BlockSpec((B, D), lambda p, b: (0, 0)),
            pl.BlockSpec((B, D), lambda p, b: (0, 0)),
            pl.BlockSpec((B, D), lambda p, b: (0, 0)),
            pl.BlockSpec((B, D), lambda p, b: (0, 0)),
        ],
        out_specs=[
            pl.BlockSpec((1, 1), lambda p, b: (0, 0), memory_space=pltpu.SMEM),
            pl.BlockSpec((1, 1), lambda p, b: (0, 0), memory_space=pltpu.SMEM),
            pl.BlockSpec((1, 1), lambda p, b: (0, 0), memory_space=pltpu.SMEM),
            pl.BlockSpec((1, 1), lambda p, b: (0, 0), memory_space=pltpu.SMEM),
        ],
        out_shape=[scalar, scalar, scalar, scalar],
    )(z1s, z2s, z2s, ue, pe, ne, ue0, pe0, ne0)


def _propagate(user_table, item_table, src, dst, w, perturbed, key):
    all_emb = jnp.concatenate([user_table, item_table], axis=0)
    acc = jnp.zeros((N, D), jnp.float32)
    for l in range(LAYERS):
        msg = all_emb[src] * w[:, None]
        all_emb = jnp.zeros((N, D), jnp.float32).at[dst].add(msg)
        if perturbed:
            key, sk = jax.random.split(key)
            noise = jax.random.normal(sk, (N, D), jnp.float32)
            all_emb = all_emb + jnp.sign(all_emb) * _l2norm(noise) * EPS
        acc = acc + all_emb
    light_out = acc / LAYERS
    return light_out[:U], light_out[U:]


def kernel(user_table, item_table, edge_u, edge_i, user_id, item_id, neg_item_id):
    deg_u = jnp.maximum(jnp.zeros(U, jnp.float32).at[edge_u].add(1.0), 1.0)
    deg_i = jnp.maximum(jnp.zeros(I, jnp.float32).at[edge_i].add(1.0), 1.0)
    w = 1.0 / jnp.sqrt(deg_u[edge_u] * deg_i[edge_i])
    src = jnp.concatenate([edge_u, edge_i + U])
    dst = jnp.concatenate([edge_i + U, edge_u])
    w2 = jnp.concatenate([w, w])

    fu, fi = _propagate(user_table, item_table, src, dst, w2, False, jax.random.key(1))
    fu1, fi1 = _propagate(user_table, item_table, src, dst, w2, True, jax.random.key(2))
    fu2, fi2 = _propagate(user_table, item_table, src, dst, w2, True, jax.random.key(3))

    z1s = jnp.stack([fu1[user_id], fi1[item_id]])
    z2s = jnp.stack([fu2[user_id], fi2[item_id]])
    ue = fu[user_id]
    pe = fi[item_id]
    ne = fi[neg_item_id]
    ue0 = user_table[user_id]
    pe0 = item_table[item_id]
    ne0 = item_table[neg_item_id]

    sslu, ssli, bprs, regs = _loss_parts(z1s, z2s, ue, pe, ne, ue0, pe0, ne0)
    bpr = -bprs[0, 0] / B
    ssl = (sslu[0, 0] + ssli[0, 0]) / B
    reg = LMBD_REG * 0.5 * regs[0, 0] / B
    return bpr + ssl * LMBD_SSL + reg * LMBD_REG


# speedup vs baseline: 5.9393x; 5.9069x over previous
import functools

import jax
import jax.numpy as jnp
from jax import lax
from jax.experimental import pallas as pl
from jax.experimental.pallas import tpu as pltpu

from jax.experimental.pallas import tpu_sc as plsc

U = 25000
I = 25000
N = U + I
D = 64
E = 400000
B = 4096
LAYERS = 3
EPS = 0.1
TAU = 0.2
LMBD_SSL = 0.1
LMBD_REG = 1e-4

BR = 512
NB = B // BR


def _l2norm(x):
    return x / jnp.maximum(jnp.linalg.norm(x, axis=-1, keepdims=True), 1e-12)


def _loss_body(z1_ref, z2full_ref, z2blk_ref, ue_ref, pe_ref, ne_ref,
               ue0_ref, pe0_ref, ne0_ref,
               sslu_ref, ssli_ref, bpr_ref, reg_ref):
    p = pl.program_id(0)
    b = pl.program_id(1)

    @pl.when(jnp.logical_and(p == 0, b == 0))
    def _init():
        ue = ue_ref[...]
        pe = pe_ref[...]
        ne = ne_ref[...]
        s = jnp.sum(ue * (pe - ne), axis=-1)
        ls = jnp.minimum(s, 0.0) - jnp.log1p(jnp.exp(-jnp.abs(s)))
        bpr_ref[0, 0] = jnp.sum(ls)
        reg_ref[0, 0] = (jnp.sum(ue0_ref[...] ** 2) + jnp.sum(pe0_ref[...] ** 2)
                         + jnp.sum(ne0_ref[...] ** 2))
        sslu_ref[0, 0] = 0.0
        ssli_ref[0, 0] = 0.0

    z1 = z1_ref[0]
    z2f = z2full_ref[0]
    z2b = z2blk_ref[0]
    n1 = z1 / jnp.maximum(jnp.sqrt(jnp.sum(z1 * z1, -1, keepdims=True)), 1e-12)
    n2f = z2f / jnp.maximum(jnp.sqrt(jnp.sum(z2f * z2f, -1, keepdims=True)), 1e-12)
    n2b = z2b / jnp.maximum(jnp.sqrt(jnp.sum(z2b * z2b, -1, keepdims=True)), 1e-12)
    pos = jnp.sum(n1 * n2b, axis=-1) / TAU
    logits = lax.dot_general(n1, n2f, (((1,), (1,)), ((), ())),
                             preferred_element_type=jnp.float32) / TAU
    m = jnp.max(logits, axis=1)
    lse = m + jnp.log(jnp.sum(jnp.exp(logits - m[:, None]), axis=1))
    val = jnp.sum(lse - pos)

    @pl.when(p == 0)
    def _accu():
        sslu_ref[0, 0] += val

    @pl.when(p == 1)
    def _acci():
        ssli_ref[0, 0] += val


@jax.jit
def _loss_parts(z1s, z2s, ue, pe, ne, ue0, pe0, ne0):
    scalar = jax.ShapeDtypeStruct((1, 1), jnp.float32)
    smem = pl.BlockSpec(memory_space=pltpu.SMEM)
    grid = (2, NB)
    return pl.pallas_call(
        _loss_body,
        grid=grid,
        in_specs=[
            pl.BlockSpec((1, BR, D), lambda p, b: (p, b, 0)),
            pl.BlockSpec((1, B, D), lambda p, b: (p, 0, 0)),
            pl.BlockSpec((1, BR, D), lambda p, b: (p, b, 0)),
            pl.BlockSpec((B, D), lambda p, b: (0, 0)),
            pl.BlockSpec((B, D), lambda p, b: (0, 0)),
            pl.BlockSpec((B, D), lambda p, b: (0, 0)),
            pl.BlockSpec((B, D), lambda p, b: (0, 0)),
            pl.BlockSpec((B, D), lambda p, b: (0, 0)),
            pl.BlockSpec((B, D), lambda p, b: (0, 0)),
        ],
        out_specs=[
            pl.BlockSpec((1, 1), lambda p, b: (0, 0), memory_space=pltpu.SMEM),
            pl.BlockSpec((1, 1), lambda p, b: (0, 0), memory_space=pltpu.SMEM),
            pl.BlockSpec((1, 1), lambda p, b: (0, 0), memory_space=pltpu.SMEM),
            pl.BlockSpec((1, 1), lambda p, b: (0, 0), memory_space=pltpu.SMEM),
        ],
        out_shape=[scalar, scalar, scalar, scalar],
    )(z1s, z2s, z2s, ue, pe, ne, ue0, pe0, ne0)


ET = E // 16
CH = 200
NCH = ET // CH
ZCH0 = 1000
ZCH = U // ZCH0


@functools.partial(
    pl.kernel,
    out_type=jax.ShapeDtypeStruct((N, D), jnp.float32),
    mesh=plsc.VectorSubcoreMesh(core_axis_name="c", subcore_axis_name="s"),
    scratch_types=[
        pltpu.VMEM((CH,), jnp.int32),
        pltpu.VMEM((CH,), jnp.int32),
        pltpu.VMEM((CH, D), jnp.float32),
        pltpu.VMEM_SHARED((U, D), jnp.float32),
        pltpu.SemaphoreType.DMA,
    ],
    compiler_params=pltpu.CompilerParams(use_tc_tiling_on_sc=False),
)
def _spmm(xs_hbm, src_hbm, dstl_hbm, zeros_hbm, y_hbm,
          src_v, dst_v, rows_v, slab, sem):
    c = lax.axis_index("c")
    s = lax.axis_index("s")

    for k in range(2):
        i = s + 16 * k

        @pl.when(i < ZCH)
        def _z():
            pltpu.sync_copy(zeros_hbm.at[pl.ds(i * ZCH0, ZCH0)],
                            slab.at[pl.ds(i * ZCH0, ZCH0)])

    plsc.subcore_barrier()

    edge_base = c * E + s * ET

    def body(i, carry):
        b = edge_base + i * CH
        pltpu.sync_copy(src_hbm.at[pl.ds(b, CH)], src_v)
        pltpu.sync_copy(dstl_hbm.at[pl.ds(b, CH)], dst_v)
        pltpu.async_copy(xs_hbm.at[src_v], rows_v, sem).wait()
        pltpu.sync_copy(rows_v, slab.at[dst_v], add=True)
        return carry

    lax.fori_loop(0, NCH, body, 0)
    plsc.subcore_barrier()

    out_base = (1 - c) * U
    for k in range(2):
        i = s + 16 * k

        @pl.when(i < ZCH)
        def _w():
            pltpu.sync_copy(slab.at[pl.ds(i * ZCH0, ZCH0)],
                            y_hbm.at[pl.ds(out_base + i * ZCH0, ZCH0)])


def _propagate(all_emb0, dinv, src, dstl, zeros, perturbed, key):
    x = all_emb0
    acc = jnp.zeros((N, D), jnp.float32)
    for l in range(LAYERS):
        xs = x * dinv[:, None]
        y = _spmm(xs, src, dstl, zeros)
        x = y * dinv[:, None]
        if perturbed:
            key, sk = jax.random.split(key)
            noise = jax.random.normal(sk, (N, D), jnp.float32)
            x = x + jnp.sign(x) * _l2norm(noise) * EPS
        acc = acc + x
    light_out = acc / LAYERS
    return light_out[:U], light_out[U:]


def kernel(user_table, item_table, edge_u, edge_i, user_id, item_id, neg_item_id):
    deg_u = jnp.maximum(jnp.zeros(U, jnp.float32).at[edge_u].add(1.0), 1.0)
    deg_i = jnp.maximum(jnp.zeros(I, jnp.float32).at[edge_i].add(1.0), 1.0)
    dinv = 1.0 / jnp.sqrt(jnp.concatenate([deg_u, deg_i]))
    src = jnp.concatenate([edge_u, edge_i + U]).astype(jnp.int32)
    dstl = jnp.concatenate([edge_i, edge_u]).astype(jnp.int32)
    zeros = jnp.zeros((U, D), jnp.float32)
    all_emb0 = jnp.concatenate([user_table, item_table], axis=0)

    fu, fi = _propagate(all_emb0, dinv, src, dstl, zeros, False, jax.random.key(1))
    fu1, fi1 = _propagate(all_emb0, dinv, src, dstl, zeros, True, jax.random.key(2))
    fu2, fi2 = _propagate(all_emb0, dinv, src, dstl, zeros, True, jax.random.key(3))

    z1s = jnp.stack([fu1[user_id], fi1[item_id]])
    z2s = jnp.stack([fu2[user_id], fi2[item_id]])
    ue = fu[user_id]
    pe = fi[item_id]
    ne = fi[neg_item_id]
    ue0 = user_table[user_id]
    pe0 = item_table[item_id]
    ne0 = item_table[neg_item_id]

    sslu, ssli, bprs, regs = _loss_parts(z1s, z2s, ue, pe, ne, ue0, pe0, ne0)
    bpr = -bprs[0, 0] / B
    ssl = (sslu[0, 0] + ssli[0, 0]) / B
    reg = LMBD_REG * 0.5 * regs[0, 0] / B
    return bpr + ssl * LMBD_SSL + reg * LMBD_REG
